# baseline (speedup 1.0000x reference)
import functools

import jax
import jax.numpy as jnp
from jax import lax
from jax.experimental import pallas as pl
from jax.experimental.pallas import tpu as pltpu
from jax.experimental.pallas import tpu_sc as plsc

K_SEL = 64
NKEYS = 100000
D = 128
NQ = 256
TILE = 2048
NTILES = 49
NPAD = NTILES * TILE
BLK = 128
NBLK = NPAD // BLK
NB_CAP = 96
FCAP = 336
FCHUNKS = FCAP // 16
INT_MIN = -2147483648

NC = 2
NS = 16
NW = NC * NS
ROWS_PER_W = NQ // NW

def _gumbel_tile(j):
    r = lax.broadcasted_iota(jnp.uint32, (NQ, TILE), 0)
    c = lax.broadcasted_iota(jnp.uint32, (NQ, TILE), 1)
    e_cnt = r * jnp.uint32(NKEYS) + j.astype(jnp.uint32) * jnp.uint32(TILE) + c

    ks0 = jnp.uint32(0)
    ks1 = jnp.uint32(1)
    ks2 = jnp.uint32(0x1BD11BDA) ^ ks0 ^ ks1
    ks = (ks0, ks1, ks2)
    rots = ((13, 15, 26, 6), (17, 29, 16, 24))
    x0 = jnp.zeros_like(e_cnt) + ks[0]
    x1 = e_cnt + ks[1]
    for i in range(5):
        for rot in rots[i % 2]:
            x0 = x0 + x1
            x1 = (x1 << jnp.uint32(rot)) | (x1 >> jnp.uint32(32 - rot))
            x1 = x1 ^ x0
        x0 = x0 + ks[(i + 1) % 3]
        x1 = x1 + ks[(i + 2) % 3] + jnp.uint32(i + 1)
    bits = x0 ^ x1

    u = lax.bitcast_convert_type(
        (bits >> jnp.uint32(9)) | jnp.uint32(0x3F800000), jnp.float32) - 1.0
    u = jnp.maximum(jnp.float32(0.0), u)
    e = -jnp.log1p(-u)
    return -jnp.log(e + 1e-20)


def _score_kernel(q_ref, k_ref, s_ref, m_ref):
    j = pl.program_id(0)
    q = q_ref[...]
    k = k_ref[...]
    s = jax.lax.dot_general(q, k, (((1,), (1,)), ((), ())),
                            preferred_element_type=jnp.float32)
    s = s + _gumbel_tile(j)
    col = j * TILE + jax.lax.broadcasted_iota(jnp.int32, s.shape, 1)
    s = jnp.where(col < NKEYS, s, -jnp.inf)
    s3 = s.reshape(NQ, TILE // BLK, BLK)
    s_ref[...] = s3
    m_ref[0] = jnp.max(s3, axis=-1)


def _scores_and_blockmax(q2, keys):
    return pl.pallas_call(
        _score_kernel,
        grid=(NTILES,),
        in_specs=[
            pl.BlockSpec((NQ, D), lambda j: (0, 0)),
            pl.BlockSpec((TILE, D), lambda j: (j, 0)),
        ],
        out_specs=[
            pl.BlockSpec((NQ, TILE // BLK, BLK), lambda j: (0, j, 0)),
            pl.BlockSpec((1, NQ, TILE // BLK), lambda j: (j, 0, 0)),
        ],
        out_shape=[
            jax.ShapeDtypeStruct((NQ, NBLK, BLK), jnp.float32),
            jax.ShapeDtypeStruct((NTILES, NQ, TILE // BLK), jnp.float32),
        ],
    )(q2, keys)


def _f32_sortkey(v):
    b = lax.bitcast_convert_type(v, jnp.int32)
    return b ^ (lax.shift_right_arithmetic(b, 31) & jnp.int32(0x7FFFFFFF))


def _splat_i32(x):
    return jnp.broadcast_to(jnp.int32(x), (16,))


def _popcnt(mask):
    return jnp.max(plsc.all_reduce_population_count(mask))


def _bisect_kth(buf, nchunks, kth, unroll, nbits=31):
    def count_ge(c):
        def cc(i, acc):
            k = buf[pl.ds(i * 16, 16)]
            return acc + jnp.where(k >= c, jnp.int32(1), jnp.int32(0))
        acc = lax.fori_loop(0, nchunks, cc, jnp.zeros((16,), jnp.int32),
                            unroll=unroll)
        return jnp.sum(acc)

    t0 = jnp.where(count_ge(jnp.int32(0)) >= kth, jnp.int32(0),
                   jnp.int32(INT_MIN))

    def step(bi, t):
        c = t | lax.shift_left(jnp.int32(1), jnp.int32(30) - bi)
        return jnp.where(count_ge(c) >= kth, c, t)

    return lax.fori_loop(0, nbits, step, t0)


def _sc_select(scores_t, m_rows, keys, values):
    mesh = plsc.VectorSubcoreMesh(core_axis_name="c", subcore_axis_name="s")

    @functools.partial(
        pl.kernel,
        out_type=[jax.ShapeDtypeStruct((NQ, D), jnp.float32),
                  jax.ShapeDtypeStruct((NQ, D), jnp.float32)],
        mesh=mesh,
        scratch_types=[
            pltpu.VMEM((NTILES, ROWS_PER_W, 16), jnp.float32),
            pltpu.VMEM((NBLK,), jnp.int32),
            pltpu.VMEM((NB_CAP,), jnp.int32),
            pltpu.VMEM((NB_CAP, BLK), jnp.float32),
            pltpu.VMEM((FCAP,), jnp.int32),
            pltpu.VMEM((FCAP,), jnp.int32),
            pltpu.VMEM((96,), jnp.int32),
            pltpu.VMEM((K_SEL,), jnp.int32),
            pltpu.VMEM((K_SEL,), jnp.int32),
            pltpu.VMEM((K_SEL, D), jnp.float32),
            pltpu.VMEM((K_SEL, D), jnp.float32),
            pltpu.VMEM((ROWS_PER_W, D), jnp.float32),
            pltpu.VMEM((ROWS_PER_W, D), jnp.float32),
            pltpu.SemaphoreType.DMA,
            pltpu.SemaphoreType.DMA,
        ],
        compiler_params=pltpu.CompilerParams(needs_layout_passes=False),
    )
    def sc_kernel(scores_ref, m_ref, keys_ref, values_ref, kv_ref, vv_ref,
                  mloc, mkeys, blks, cand, kbuf, gibuf, fidxw, fidx, eqbuf,
                  krows, vrows, kacc, vacc, sem1, sem2):
        wid = lax.axis_index("s") * NC + lax.axis_index("c")
        lane = lax.iota(jnp.int32, 16)
        pltpu.sync_copy(m_ref.at[:, pl.ds(wid * ROWS_PER_W, ROWS_PER_W), :],
                        mloc)

        def do_row(r, _):
            row = wid * ROWS_PER_W + r
            base = row * NBLK
            pad_blk = base + NBLK - 1

            def tr(i, _):
                mkeys[pl.ds(i * 16, 16)] = _f32_sortkey(mloc[i, r, :])
                return 0
            lax.fori_loop(0, NBLK // 16, tr, 0, unroll=7)

            tb = _bisect_kth(mkeys, NBLK // 16, K_SEL, unroll=7, nbits=23)

            def initb(i, _):
                blks[pl.ds(i * 16, 16)] = _splat_i32(0) + pad_blk
                return 0
            lax.fori_loop(0, NB_CAP // 16, initb, 0, unroll=5)

            def selb(i, w):
                k = mkeys[pl.ds(i * 16, 16)]
                m = k >= tb
                gi = base + i * 16 + lane
                wc = jnp.minimum(w, jnp.int32(NB_CAP - 16))
                plsc.store_compressed(blks.at[pl.ds(wc, 16)], gi, mask=m)
                return w + _popcnt(m)
            nb = lax.fori_loop(0, NBLK // 16, selb, jnp.int32(0))
            nb = jnp.minimum(nb, jnp.int32(NB_CAP))

            pltpu.async_copy(scores_ref.at[blks], cand, sem1).wait()

            def initk(i, _):
                kbuf[pl.ds(i * 16, 16)] = _splat_i32(INT_MIN)
                return 0
            lax.fori_loop(0, FCHUNKS, initk, 0, unroll=7)

            def frow(rr, w):
                bid = plsc.load_gather(blks, [jnp.broadcast_to(rr, (16,))])
                gbase = (bid - base) * BLK

                def fchunk(c, w):
                    v = cand[rr, pl.ds(c * 16, 16)]
                    k = _f32_sortkey(v)
                    m = k >= tb
                    gi = gbase + c * 16 + lane
                    wc = jnp.minimum(w, jnp.int32(FCAP - 16))
                    plsc.store_compressed(kbuf.at[pl.ds(wc, 16)], k, mask=m)
                    plsc.store_compressed(gibuf.at[pl.ds(wc, 16)], gi, mask=m)
                    return w + _popcnt(m)
                return lax.fori_loop(0, BLK // 16, fchunk, w, unroll=8)
            lax.fori_loop(0, nb, frow, jnp.int32(0))

            t64 = _bisect_kth(kbuf, FCHUNKS, K_SEL, unroll=7)

            def spass(i, carry):
                w1, w2 = carry
                k = kbuf[pl.ds(i * 16, 16)]
                gi = gibuf[pl.ds(i * 16, 16)]
                m1 = k > t64
                m2 = k == t64
                w1c = jnp.minimum(w1, jnp.int32(80))
                plsc.store_compressed(fidxw.at[pl.ds(w1c, 16)], gi, mask=m1)
                w2c = jnp.minimum(w2, jnp.int32(K_SEL - 16))
                plsc.store_compressed(eqbuf.at[pl.ds(w2c, 16)], gi, mask=m2)
                return w1 + _popcnt(m1), w2 + _popcnt(m2)
            w1, _ = lax.fori_loop(0, FCHUNKS, spass,
                                  (jnp.int32(0), jnp.int32(0)))
            need = jnp.int32(K_SEL) - w1

            def cpy(c, _):
                gi = eqbuf[pl.ds(c * 16, 16)]
                m = (c * 16 + lane) < need
                off = jnp.minimum(w1 + c * 16, jnp.int32(80))
                plsc.store_compressed(fidxw.at[pl.ds(off, 16)], gi, mask=m)
                return 0
            lax.fori_loop(0, K_SEL // 16, cpy, 0)
            for c in range(K_SEL // 16):
                fidx[pl.ds(c * 16, 16)] = fidxw[pl.ds(c * 16, 16)]

            ck = pltpu.async_copy(keys_ref.at[fidx], krows, sem1)
            cv = pltpu.async_copy(values_ref.at[fidx], vrows, sem2)
            ck.wait()
            cv.wait()

            zero = jnp.zeros((16,), jnp.float32)

            def acc_row(rr, carry):
                return tuple(
                    carry[c] + krows[rr, pl.ds(c * 16, 16)] for c in range(8)
                ) + tuple(
                    carry[8 + c] + vrows[rr, pl.ds(c * 16, 16)]
                    for c in range(8)
                )
            sums = lax.fori_loop(0, K_SEL, acc_row, (zero,) * 16)
            scale = jnp.float32(1.0 / K_SEL)
            for c in range(8):
                kacc[r, pl.ds(c * 16, 16)] = sums[c] * scale
                vacc[r, pl.ds(c * 16, 16)] = sums[8 + c] * scale
            return 0

        lax.fori_loop(0, ROWS_PER_W, do_row, 0)
        pltpu.sync_copy(kacc, kv_ref.at[pl.ds(wid * ROWS_PER_W, ROWS_PER_W)])
        pltpu.sync_copy(vacc, vv_ref.at[pl.ds(wid * ROWS_PER_W, ROWS_PER_W)])

    return sc_kernel(scores_t, m_rows, keys, values)


def kernel(query, keys, values):
    B, L, _ = query.shape
    q2 = query.reshape(B * L, D)
    scores, m3 = _scores_and_blockmax(q2, keys)
    scores_t = scores.reshape(NQ * NBLK, BLK)
    kv, vv = _sc_select(scores_t, m3, keys, values)
    return kv.reshape(B, L, D), vv.reshape(B, L, D)


# speedup vs baseline: 1.0899x; 1.0899x over previous
import functools

import jax
import jax.numpy as jnp
from jax import lax
from jax.experimental import pallas as pl
from jax.experimental.pallas import tpu as pltpu
from jax.experimental.pallas import tpu_sc as plsc

K_SEL = 64
NKEYS = 100000
D = 128
NQ = 256
TILE = 2048
NTILES = 49
NPAD = NTILES * TILE
BLK = 128
NBLK = NPAD // BLK
NB_CAP = 96
FCAP = 336
FCHUNKS = FCAP // 16
INT_MIN = -2147483648

NC = 2
NS = 16
NW = NC * NS
ROWS_PER_W = NQ // NW

def _gumbel_tile(j, nq, row0):
    r = lax.broadcasted_iota(jnp.uint32, (nq, TILE), 0) + jnp.uint32(row0)
    c = lax.broadcasted_iota(jnp.uint32, (nq, TILE), 1)
    e_cnt = r * jnp.uint32(NKEYS) + j.astype(jnp.uint32) * jnp.uint32(TILE) + c

    ks0 = jnp.uint32(0)
    ks1 = jnp.uint32(1)
    ks2 = jnp.uint32(0x1BD11BDA) ^ ks0 ^ ks1
    ks = (ks0, ks1, ks2)
    rots = ((13, 15, 26, 6), (17, 29, 16, 24))
    x0 = jnp.zeros_like(e_cnt) + ks[0]
    x1 = e_cnt + ks[1]
    for i in range(5):
        for rot in rots[i % 2]:
            x0 = x0 + x1
            x1 = (x1 << jnp.uint32(rot)) | (x1 >> jnp.uint32(32 - rot))
            x1 = x1 ^ x0
        x0 = x0 + ks[(i + 1) % 3]
        x1 = x1 + ks[(i + 2) % 3] + jnp.uint32(i + 1)
    bits = x0 ^ x1

    u = lax.bitcast_convert_type(
        (bits >> jnp.uint32(9)) | jnp.uint32(0x3F800000), jnp.float32) - 1.0
    u = jnp.maximum(jnp.float32(0.0), u)
    e = -jnp.log1p(-u)
    return -jnp.log(e + 1e-20)


def _score_kernel(q_ref, k_ref, s_ref, m_ref, *, nq, row0):
    j = pl.program_id(0)
    q = q_ref[...]
    k = k_ref[...]
    s = jax.lax.dot_general(q, k, (((1,), (1,)), ((), ())),
                            preferred_element_type=jnp.float32)
    s = s + _gumbel_tile(j, nq, row0)
    col = j * TILE + jax.lax.broadcasted_iota(jnp.int32, s.shape, 1)
    s = jnp.where(col < NKEYS, s, -jnp.inf)
    s3 = s.reshape(nq, TILE // BLK, BLK)
    s_ref[...] = s3
    m_ref[0] = jnp.max(s3, axis=-1)


def _scores_and_blockmax(q2, keys, nq, row0):
    return pl.pallas_call(
        functools.partial(_score_kernel, nq=nq, row0=row0),
        grid=(NTILES,),
        in_specs=[
            pl.BlockSpec((nq, D), lambda j: (0, 0)),
            pl.BlockSpec((TILE, D), lambda j: (j, 0)),
        ],
        out_specs=[
            pl.BlockSpec((nq, TILE // BLK, BLK), lambda j: (0, j, 0)),
            pl.BlockSpec((1, nq, TILE // BLK), lambda j: (j, 0, 0)),
        ],
        out_shape=[
            jax.ShapeDtypeStruct((nq, NBLK, BLK), jnp.float32),
            jax.ShapeDtypeStruct((NTILES, nq, TILE // BLK), jnp.float32),
        ],
    )(q2, keys)


def _f32_sortkey(v):
    b = lax.bitcast_convert_type(v, jnp.int32)
    return b ^ (lax.shift_right_arithmetic(b, 31) & jnp.int32(0x7FFFFFFF))


def _splat_i32(x):
    return jnp.broadcast_to(jnp.int32(x), (16,))


def _popcnt(mask):
    return jnp.max(plsc.all_reduce_population_count(mask))


def _bisect_kth(buf, nchunks, kth, unroll, nbits=31):
    def count_ge(c):
        def cc(i, acc):
            k = buf[pl.ds(i * 16, 16)]
            return acc + jnp.where(k >= c, jnp.int32(1), jnp.int32(0))
        acc = lax.fori_loop(0, nchunks, cc, jnp.zeros((16,), jnp.int32),
                            unroll=unroll)
        return jnp.sum(acc)

    t0 = jnp.where(count_ge(jnp.int32(0)) >= kth, jnp.int32(0),
                   jnp.int32(INT_MIN))

    def step(bi, t):
        c = t | lax.shift_left(jnp.int32(1), jnp.int32(30) - bi)
        return jnp.where(count_ge(c) >= kth, c, t)

    return lax.fori_loop(0, nbits, step, t0)


def _sc_select(scores_t, m_rows, keys, values, nq):
    mesh = plsc.VectorSubcoreMesh(core_axis_name="c", subcore_axis_name="s")
    rpw = nq // NW

    @functools.partial(
        pl.kernel,
        out_type=[jax.ShapeDtypeStruct((nq, D), jnp.float32),
                  jax.ShapeDtypeStruct((nq, D), jnp.float32)],
        mesh=mesh,
        scratch_types=[
            pltpu.VMEM((NTILES, rpw, 16), jnp.float32),
            pltpu.VMEM((NBLK,), jnp.int32),
            pltpu.VMEM((NB_CAP,), jnp.int32),
            pltpu.VMEM((NB_CAP, BLK), jnp.float32),
            pltpu.VMEM((FCAP,), jnp.int32),
            pltpu.VMEM((FCAP,), jnp.int32),
            pltpu.VMEM((96,), jnp.int32),
            pltpu.VMEM((K_SEL,), jnp.int32),
            pltpu.VMEM((K_SEL,), jnp.int32),
            pltpu.VMEM((K_SEL, D), jnp.float32),
            pltpu.VMEM((K_SEL, D), jnp.float32),
            pltpu.VMEM((rpw, D), jnp.float32),
            pltpu.VMEM((rpw, D), jnp.float32),
            pltpu.SemaphoreType.DMA,
            pltpu.SemaphoreType.DMA,
        ],
        compiler_params=pltpu.CompilerParams(needs_layout_passes=False),
    )
    def sc_kernel(scores_ref, m_ref, keys_ref, values_ref, kv_ref, vv_ref,
                  mloc, mkeys, blks, cand, kbuf, gibuf, fidxw, fidx, eqbuf,
                  krows, vrows, kacc, vacc, sem1, sem2):
        wid = lax.axis_index("s") * NC + lax.axis_index("c")
        lane = lax.iota(jnp.int32, 16)
        pltpu.sync_copy(m_ref.at[:, pl.ds(wid * rpw, rpw), :], mloc)

        def do_row(r, _):
            row = wid * rpw + r
            base = row * NBLK
            pad_blk = base + NBLK - 1

            def tr(i, _):
                mkeys[pl.ds(i * 16, 16)] = _f32_sortkey(mloc[i, r, :])
                return 0
            lax.fori_loop(0, NBLK // 16, tr, 0, unroll=7)

            tb = _bisect_kth(mkeys, NBLK // 16, K_SEL, unroll=7, nbits=23)

            def initb(i, _):
                blks[pl.ds(i * 16, 16)] = _splat_i32(0) + pad_blk
                return 0
            lax.fori_loop(0, NB_CAP // 16, initb, 0, unroll=5)

            def selb(i, w):
                k = mkeys[pl.ds(i * 16, 16)]
                m = k >= tb
                gi = base + i * 16 + lane
                wc = jnp.minimum(w, jnp.int32(NB_CAP - 16))
                plsc.store_compressed(blks.at[pl.ds(wc, 16)], gi, mask=m)
                return w + _popcnt(m)
            nb = lax.fori_loop(0, NBLK // 16, selb, jnp.int32(0))
            nb = jnp.minimum(nb, jnp.int32(NB_CAP))

            pltpu.async_copy(scores_ref.at[blks], cand, sem1).wait()

            def initk(i, _):
                kbuf[pl.ds(i * 16, 16)] = _splat_i32(INT_MIN)
                return 0
            lax.fori_loop(0, FCHUNKS, initk, 0, unroll=7)

            def frow(rr, w):
                bid = plsc.load_gather(blks, [jnp.broadcast_to(rr, (16,))])
                gbase = (bid - base) * BLK

                def fchunk(c, w):
                    v = cand[rr, pl.ds(c * 16, 16)]
                    k = _f32_sortkey(v)
                    m = k >= tb
                    gi = gbase + c * 16 + lane
                    wc = jnp.minimum(w, jnp.int32(FCAP - 16))
                    plsc.store_compressed(kbuf.at[pl.ds(wc, 16)], k, mask=m)
                    plsc.store_compressed(gibuf.at[pl.ds(wc, 16)], gi, mask=m)
                    return w + _popcnt(m)
                return lax.fori_loop(0, BLK // 16, fchunk, w, unroll=8)
            lax.fori_loop(0, nb, frow, jnp.int32(0))

            t64 = _bisect_kth(kbuf, FCHUNKS, K_SEL, unroll=7)

            def spass(i, carry):
                w1, w2 = carry
                k = kbuf[pl.ds(i * 16, 16)]
                gi = gibuf[pl.ds(i * 16, 16)]
                m1 = k > t64
                m2 = k == t64
                w1c = jnp.minimum(w1, jnp.int32(80))
                plsc.store_compressed(fidxw.at[pl.ds(w1c, 16)], gi, mask=m1)
                w2c = jnp.minimum(w2, jnp.int32(K_SEL - 16))
                plsc.store_compressed(eqbuf.at[pl.ds(w2c, 16)], gi, mask=m2)
                return w1 + _popcnt(m1), w2 + _popcnt(m2)
            w1, _ = lax.fori_loop(0, FCHUNKS, spass,
                                  (jnp.int32(0), jnp.int32(0)))
            need = jnp.int32(K_SEL) - w1

            def cpy(c, _):
                gi = eqbuf[pl.ds(c * 16, 16)]
                m = (c * 16 + lane) < need
                off = jnp.minimum(w1 + c * 16, jnp.int32(80))
                plsc.store_compressed(fidxw.at[pl.ds(off, 16)], gi, mask=m)
                return 0
            lax.fori_loop(0, K_SEL // 16, cpy, 0)
            for c in range(K_SEL // 16):
                fidx[pl.ds(c * 16, 16)] = fidxw[pl.ds(c * 16, 16)]

            ck = pltpu.async_copy(keys_ref.at[fidx], krows, sem1)
            cv = pltpu.async_copy(values_ref.at[fidx], vrows, sem2)
            ck.wait()
            cv.wait()

            zero = jnp.zeros((16,), jnp.float32)

            def acc_row(rr, carry):
                return tuple(
                    carry[c] + krows[rr, pl.ds(c * 16, 16)] for c in range(8)
                ) + tuple(
                    carry[8 + c] + vrows[rr, pl.ds(c * 16, 16)]
                    for c in range(8)
                )
            sums = lax.fori_loop(0, K_SEL, acc_row, (zero,) * 16)
            scale = jnp.float32(1.0 / K_SEL)
            for c in range(8):
                kacc[r, pl.ds(c * 16, 16)] = sums[c] * scale
                vacc[r, pl.ds(c * 16, 16)] = sums[8 + c] * scale
            return 0

        lax.fori_loop(0, rpw, do_row, 0)
        pltpu.sync_copy(kacc, kv_ref.at[pl.ds(wid * rpw, rpw)])
        pltpu.sync_copy(vacc, vv_ref.at[pl.ds(wid * rpw, rpw)])

    return sc_kernel(scores_t, m_rows, keys, values)


def kernel(query, keys, values):
    B, L, _ = query.shape
    q2 = query.reshape(B * L, D)
    halves = []
    nh = NQ // 2
    for h in range(2):
        qh = lax.slice_in_dim(q2, h * nh, (h + 1) * nh, axis=0)
        scores, m3 = _scores_and_blockmax(qh, keys, nh, h * nh)
        scores_t = scores.reshape(nh * NBLK, BLK)
        halves.append(_sc_select(scores_t, m3, keys, values, nh))
    kv = jnp.concatenate([halves[0][0], halves[1][0]], axis=0)
    vv = jnp.concatenate([halves[0][1], halves[1][1]], axis=0)
    return kv.reshape(B, L, D), vv.reshape(B, L, D)


# speedup vs baseline: 1.1044x; 1.0133x over previous
import functools

import jax
import jax.numpy as jnp
from jax import lax
from jax.experimental import pallas as pl
from jax.experimental.pallas import tpu as pltpu
from jax.experimental.pallas import tpu_sc as plsc

K_SEL = 64
NKEYS = 100000
D = 128
NQ = 256
TILE = 2048
NTILES = 49
NPAD = NTILES * TILE
BLK = 128
NBLK = NPAD // BLK
NB_CAP = 96
FCAP = 336
FCHUNKS = FCAP // 16
INT_MIN = -2147483648

NC = 2
NS = 16
NW = NC * NS
ROWS_PER_W = NQ // NW

def _gumbel_tile(j, nq, row0):
    r = lax.broadcasted_iota(jnp.uint32, (nq, TILE), 0) + jnp.uint32(row0)
    c = lax.broadcasted_iota(jnp.uint32, (nq, TILE), 1)
    e_cnt = r * jnp.uint32(NKEYS) + j.astype(jnp.uint32) * jnp.uint32(TILE) + c

    ks0 = jnp.uint32(0)
    ks1 = jnp.uint32(1)
    ks2 = jnp.uint32(0x1BD11BDA) ^ ks0 ^ ks1
    ks = (ks0, ks1, ks2)
    rots = ((13, 15, 26, 6), (17, 29, 16, 24))
    x0 = jnp.zeros_like(e_cnt) + ks[0]
    x1 = e_cnt + ks[1]
    for i in range(5):
        for rot in rots[i % 2]:
            x0 = x0 + x1
            x1 = (x1 << jnp.uint32(rot)) | (x1 >> jnp.uint32(32 - rot))
            x1 = x1 ^ x0
        x0 = x0 + ks[(i + 1) % 3]
        x1 = x1 + ks[(i + 2) % 3] + jnp.uint32(i + 1)
    bits = x0 ^ x1

    u = lax.bitcast_convert_type(
        (bits >> jnp.uint32(9)) | jnp.uint32(0x3F800000), jnp.float32) - 1.0
    u = jnp.maximum(jnp.float32(0.0), u)
    e = -jnp.log1p(-u)
    return -jnp.log(e + 1e-20)


def _score_kernel(q_ref, k_ref, s_ref, m_ref, *, nq, row0):
    j = pl.program_id(0)
    q = q_ref[...]
    k = k_ref[...]
    s = jax.lax.dot_general(q, k, (((1,), (1,)), ((), ())),
                            preferred_element_type=jnp.float32)
    s = s + _gumbel_tile(j, nq, row0)
    col = j * TILE + jax.lax.broadcasted_iota(jnp.int32, s.shape, 1)
    s = jnp.where(col < NKEYS, s, -jnp.inf)
    s3 = s.reshape(nq, TILE // BLK, BLK)
    s_ref[...] = s3
    m_ref[0] = jnp.max(s3, axis=-1)


def _scores_and_blockmax(q2, keys, nq, row0):
    return pl.pallas_call(
        functools.partial(_score_kernel, nq=nq, row0=row0),
        grid=(NTILES,),
        in_specs=[
            pl.BlockSpec((nq, D), lambda j: (0, 0)),
            pl.BlockSpec((TILE, D), lambda j: (j, 0)),
        ],
        out_specs=[
            pl.BlockSpec((nq, TILE // BLK, BLK), lambda j: (0, j, 0)),
            pl.BlockSpec((1, nq, TILE // BLK), lambda j: (j, 0, 0)),
        ],
        out_shape=[
            jax.ShapeDtypeStruct((nq, NBLK, BLK), jnp.float32),
            jax.ShapeDtypeStruct((NTILES, nq, TILE // BLK), jnp.float32),
        ],
    )(q2, keys)


def _f32_sortkey(v):
    b = lax.bitcast_convert_type(v, jnp.int32)
    return b ^ (lax.shift_right_arithmetic(b, 31) & jnp.int32(0x7FFFFFFF))


def _splat_i32(x):
    return jnp.broadcast_to(jnp.int32(x), (16,))


def _popcnt(mask):
    return jnp.max(plsc.all_reduce_population_count(mask))


def _bisect_kth(buf, nchunks, kth, unroll, nbits=31):
    def count_ge(c):
        def cc(i, acc):
            k = buf[pl.ds(i * 16, 16)]
            return acc + jnp.where(k >= c, jnp.int32(1), jnp.int32(0))
        acc = lax.fori_loop(0, nchunks, cc, jnp.zeros((16,), jnp.int32),
                            unroll=unroll)
        return jnp.sum(acc)

    t0 = jnp.where(count_ge(jnp.int32(0)) >= kth, jnp.int32(0),
                   jnp.int32(INT_MIN))

    def step(bi, t):
        c = t | lax.shift_left(jnp.int32(1), jnp.int32(30) - bi)
        return jnp.where(count_ge(c) >= kth, c, t)

    return lax.fori_loop(0, nbits, step, t0)


def _sc_select(scores_t, m_rows, keys, values, nq):
    mesh = plsc.VectorSubcoreMesh(core_axis_name="c", subcore_axis_name="s")
    rpw = nq // NW

    @functools.partial(
        pl.kernel,
        out_type=[jax.ShapeDtypeStruct((nq, D), jnp.float32),
                  jax.ShapeDtypeStruct((nq, D), jnp.float32)],
        mesh=mesh,
        scratch_types=[
            pltpu.VMEM((NTILES, rpw, 16), jnp.float32),
            pltpu.VMEM((NBLK,), jnp.int32),
            pltpu.VMEM((NB_CAP,), jnp.int32),
            pltpu.VMEM((NB_CAP, BLK), jnp.float32),
            pltpu.VMEM((FCAP,), jnp.int32),
            pltpu.VMEM((FCAP,), jnp.int32),
            pltpu.VMEM((96,), jnp.int32),
            pltpu.VMEM((K_SEL,), jnp.int32),
            pltpu.VMEM((K_SEL,), jnp.int32),
            pltpu.VMEM((K_SEL, D), jnp.float32),
            pltpu.VMEM((K_SEL, D), jnp.float32),
            pltpu.VMEM((rpw, D), jnp.float32),
            pltpu.VMEM((rpw, D), jnp.float32),
            pltpu.SemaphoreType.DMA,
            pltpu.SemaphoreType.DMA,
        ],
        compiler_params=pltpu.CompilerParams(needs_layout_passes=False),
    )
    def sc_kernel(scores_ref, m_ref, keys_ref, values_ref, kv_ref, vv_ref,
                  mloc, mkeys, blks, cand, kbuf, gibuf, fidxw, fidx, eqbuf,
                  krows, vrows, kacc, vacc, sem1, sem2):
        wid = lax.axis_index("s") * NC + lax.axis_index("c")
        lane = lax.iota(jnp.int32, 16)
        pltpu.sync_copy(m_ref.at[:, pl.ds(wid * rpw, rpw), :], mloc)

        def do_row(r, _):
            row = wid * rpw + r
            base = row * NBLK
            pad_blk = base + NBLK - 1

            def tr(i, _):
                mkeys[pl.ds(i * 16, 16)] = _f32_sortkey(mloc[i, r, :])
                return 0
            lax.fori_loop(0, NBLK // 16, tr, 0, unroll=7)

            tb = _bisect_kth(mkeys, NBLK // 16, K_SEL, unroll=7, nbits=23)

            def initb(i, _):
                blks[pl.ds(i * 16, 16)] = _splat_i32(0) + pad_blk
                return 0
            lax.fori_loop(0, NB_CAP // 16, initb, 0, unroll=5)

            def selb(i, w):
                k = mkeys[pl.ds(i * 16, 16)]
                m = k >= tb
                gi = base + i * 16 + lane
                wc = jnp.minimum(w, jnp.int32(NB_CAP - 16))
                plsc.store_compressed(blks.at[pl.ds(wc, 16)], gi, mask=m)
                return w + _popcnt(m)
            nb = lax.fori_loop(0, NBLK // 16, selb, jnp.int32(0))
            nb = jnp.minimum(nb, jnp.int32(NB_CAP))

            pltpu.async_copy(scores_ref.at[blks], cand, sem1).wait()

            def initk(i, _):
                kbuf[pl.ds(i * 16, 16)] = _splat_i32(INT_MIN)
                return 0
            lax.fori_loop(0, FCHUNKS, initk, 0, unroll=7)

            def frow(rr, w):
                bid = plsc.load_gather(blks, [jnp.broadcast_to(rr, (16,))])
                gbase = (bid - base) * BLK

                def fchunk(c, w):
                    v = cand[rr, pl.ds(c * 16, 16)]
                    k = _f32_sortkey(v)
                    m = k >= tb
                    gi = gbase + c * 16 + lane
                    wc = jnp.minimum(w, jnp.int32(FCAP - 16))
                    plsc.store_compressed(kbuf.at[pl.ds(wc, 16)], k, mask=m)
                    plsc.store_compressed(gibuf.at[pl.ds(wc, 16)], gi, mask=m)
                    return w + _popcnt(m)
                return lax.fori_loop(0, BLK // 16, fchunk, w, unroll=8)
            lax.fori_loop(0, nb, frow, jnp.int32(0))

            t64 = _bisect_kth(kbuf, FCHUNKS, K_SEL, unroll=7)

            def spass(i, carry):
                w1, w2 = carry
                k = kbuf[pl.ds(i * 16, 16)]
                gi = gibuf[pl.ds(i * 16, 16)]
                m1 = k > t64
                m2 = k == t64
                w1c = jnp.minimum(w1, jnp.int32(80))
                plsc.store_compressed(fidxw.at[pl.ds(w1c, 16)], gi, mask=m1)
                w2c = jnp.minimum(w2, jnp.int32(K_SEL - 16))
                plsc.store_compressed(eqbuf.at[pl.ds(w2c, 16)], gi, mask=m2)
                return w1 + _popcnt(m1), w2 + _popcnt(m2)
            w1, _ = lax.fori_loop(0, FCHUNKS, spass,
                                  (jnp.int32(0), jnp.int32(0)))
            need = jnp.int32(K_SEL) - w1

            def cpy(c, _):
                gi = eqbuf[pl.ds(c * 16, 16)]
                m = (c * 16 + lane) < need
                off = jnp.minimum(w1 + c * 16, jnp.int32(80))
                plsc.store_compressed(fidxw.at[pl.ds(off, 16)], gi, mask=m)
                return 0
            lax.fori_loop(0, K_SEL // 16, cpy, 0)
            for c in range(K_SEL // 16):
                fidx[pl.ds(c * 16, 16)] = fidxw[pl.ds(c * 16, 16)]

            ck = pltpu.async_copy(keys_ref.at[fidx], krows, sem1)
            cv = pltpu.async_copy(values_ref.at[fidx], vrows, sem2)
            ck.wait()
            cv.wait()

            zero = jnp.zeros((16,), jnp.float32)

            def acc_row(rr, carry):
                return tuple(
                    carry[c] + krows[rr, pl.ds(c * 16, 16)] for c in range(8)
                ) + tuple(
                    carry[8 + c] + vrows[rr, pl.ds(c * 16, 16)]
                    for c in range(8)
                )
            sums = lax.fori_loop(0, K_SEL, acc_row, (zero,) * 16)
            scale = jnp.float32(1.0 / K_SEL)
            for c in range(8):
                kacc[r, pl.ds(c * 16, 16)] = sums[c] * scale
                vacc[r, pl.ds(c * 16, 16)] = sums[8 + c] * scale
            return 0

        lax.fori_loop(0, rpw, do_row, 0)
        pltpu.sync_copy(kacc, kv_ref.at[pl.ds(wid * rpw, rpw)])
        pltpu.sync_copy(vacc, vv_ref.at[pl.ds(wid * rpw, rpw)])

    return sc_kernel(scores_t, m_rows, keys, values)


def kernel(query, keys, values):
    B, L, _ = query.shape
    q2 = query.reshape(B * L, D)
    nsplit = 4
    parts = []
    nh = NQ // nsplit
    for h in range(nsplit):
        qh = lax.slice_in_dim(q2, h * nh, (h + 1) * nh, axis=0)
        scores, m3 = _scores_and_blockmax(qh, keys, nh, h * nh)
        scores_t = scores.reshape(nh * NBLK, BLK)
        parts.append(_sc_select(scores_t, m3, keys, values, nh))
    kv = jnp.concatenate([p[0] for p in parts], axis=0)
    vv = jnp.concatenate([p[1] for p in parts], axis=0)
    return kv.reshape(B, L, D), vv.reshape(B, L, D)
